# Initial kernel scaffold; baseline (speedup 1.0000x reference)
import jax
import jax.numpy as jnp
from jax.experimental import pallas as pl


def kernel(src, dst, t, edge_attr, memory, last_update, W_time, b_time, W1, b1, W2, b2):
    raise NotImplementedError("write your pallas kernel here")


# speedup vs baseline: 1.3390x; 1.3390x over previous
import functools

import jax
import jax.numpy as jnp
from jax import lax
from jax.experimental import pallas as pl
from jax.experimental.pallas import tpu as pltpu
from jax.experimental.pallas import tpu_sc as plsc

MEMORY_DIM = 32
TIME_DIM = 16
EDGE_FEAT_DIM = 16
HIDDEN = 128

_NC = 2
_NS = 16
_NW = _NC * _NS
_CHUNK = 128
_LANES = 16


def _sc_gather(src, dst, t, memory, last_update):
    B = src.shape[0]
    b_per_w = B // _NW
    n_chunks = b_per_w // _CHUNK
    mesh = plsc.VectorSubcoreMesh(core_axis_name="c", subcore_axis_name="s")

    @functools.partial(
        pl.kernel,
        mesh=mesh,
        compiler_params=pltpu.CompilerParams(use_tc_tiling_on_sc=False),
        out_type=(
            jax.ShapeDtypeStruct((B, MEMORY_DIM), jnp.float32),
            jax.ShapeDtypeStruct((B, MEMORY_DIM), jnp.float32),
            jax.ShapeDtypeStruct((B,), jnp.float32),
        ),
        scratch_types=[
            pltpu.VMEM((b_per_w,), jnp.int32),
            pltpu.VMEM((b_per_w,), jnp.int32),
            pltpu.VMEM((b_per_w, MEMORY_DIM), jnp.float32),
            pltpu.VMEM((b_per_w, MEMORY_DIM), jnp.float32),
            pltpu.VMEM((b_per_w,), jnp.float32),
            pltpu.VMEM((b_per_w,), jnp.float32),
            pltpu.VMEM((b_per_w,), jnp.float32),
            pltpu.SemaphoreType.DMA,
        ],
    )
    def gather_kernel(mem_hbm, lu_hbm, src_hbm, dst_hbm, t_hbm,
                      srcmem_out, dstmem_out, dt_out,
                      sidx, didx, srows, drows, slu, tv, dtv, sem):
        wid = lax.axis_index("s") * _NC + lax.axis_index("c")
        base = wid * b_per_w
        pltpu.sync_copy(src_hbm.at[pl.ds(base, b_per_w)], sidx)
        pltpu.sync_copy(dst_hbm.at[pl.ds(base, b_per_w)], didx)
        pltpu.sync_copy(t_hbm.at[pl.ds(base, b_per_w)], tv)
        copies = []
        for j in range(n_chunks):
            sl = pl.ds(j * _CHUNK, _CHUNK)
            copies.append(pltpu.async_copy(mem_hbm.at[sidx.at[sl]], srows.at[sl], sem))
            copies.append(pltpu.async_copy(mem_hbm.at[didx.at[sl]], drows.at[sl], sem))
            copies.append(pltpu.async_copy(lu_hbm.at[sidx.at[sl]], slu.at[sl], sem))
        for c in copies:
            c.wait()
        for i in range(b_per_w // _LANES):
            s = pl.ds(i * _LANES, _LANES)
            dtv[s] = tv[s] - slu[s]
        pltpu.sync_copy(srows, srcmem_out.at[pl.ds(base, b_per_w)])
        pltpu.sync_copy(drows, dstmem_out.at[pl.ds(base, b_per_w)])
        pltpu.sync_copy(dtv, dt_out.at[pl.ds(base, b_per_w)])

    return gather_kernel(memory, last_update, src, dst, t)


def _mlp_body(sm, dm, dtb, ea, wt, bt, w1, b1r, w2, b2r, out):
    enc = jnp.cos(dtb[:] * wt[:] + bt[:])
    x = jnp.concatenate([sm[:], dm[:], enc, ea[:]], axis=1)
    h = jnp.maximum(
        jnp.dot(x, w1[:], preferred_element_type=jnp.float32) + b1r[:], 0.0)
    out[:] = jnp.dot(h, w2[:], preferred_element_type=jnp.float32) + b2r[0, 0]


def _tc_mlp(src_mem, dst_mem, dt, edge_attr, W_time, b_time, W1, b1, W2, b2):
    B = src_mem.shape[0]
    BLK = 2048
    grid = (B // BLK,)
    blk = lambda r, c: pl.BlockSpec((r, c), lambda i: (i, 0))
    full = lambda r, c: pl.BlockSpec((r, c), lambda i: (0, 0))
    return pl.pallas_call(
        _mlp_body,
        grid=grid,
        in_specs=[
            blk(BLK, MEMORY_DIM),
            blk(BLK, MEMORY_DIM),
            blk(BLK, 1),
            blk(BLK, EDGE_FEAT_DIM),
            full(1, TIME_DIM),
            full(1, TIME_DIM),
            full(2 * MEMORY_DIM + TIME_DIM + EDGE_FEAT_DIM, HIDDEN),
            full(1, HIDDEN),
            full(HIDDEN, 1),
            full(1, 1),
        ],
        out_specs=blk(BLK, 1),
        out_shape=jax.ShapeDtypeStruct((B, 1), jnp.float32),
    )(src_mem, dst_mem, dt, edge_attr, W_time, b_time, W1, b1, W2, b2)


def kernel(src, dst, t, edge_attr, memory, last_update,
           W_time, b_time, W1, b1, W2, b2):
    B = src.shape[0]
    src_mem, dst_mem, dt = _sc_gather(
        src.astype(jnp.int32), dst.astype(jnp.int32), t, memory, last_update)
    return _tc_mlp(
        src_mem, dst_mem, dt.reshape(B, 1), edge_attr.astype(jnp.float32),
        W_time.reshape(1, TIME_DIM), b_time.reshape(1, TIME_DIM),
        W1, b1.reshape(1, HIDDEN), W2, b2.reshape(1, 1))
